# baseline (speedup 1.0000x reference)
import jax
import jax.numpy as jnp
from jax import lax
from jax.experimental import pallas as pl
from jax.experimental.pallas import tpu as pltpu
from jax.experimental.pallas import tpu_sc as plsc

MAX_LEN = 4096
D_MODEL = 1024
BATCH = 4

NUM_CORES = 2
NUM_SUBCORES = 16
NUM_WORKERS = NUM_CORES * NUM_SUBCORES
ROWS_PER_WORKER = MAX_LEN // NUM_WORKERS
CHUNK = 32
NUM_CHUNKS = ROWS_PER_WORKER // CHUNK


def _sc_broadcast(table_hbm, out_hbm, buf0, buf1, gsem, ssem):
    wid = lax.axis_index("s") * NUM_CORES + lax.axis_index("c")
    base = wid * ROWS_PER_WORKER
    bufs = (buf0, buf1)

    def gather(c):
        row = base + c * CHUNK
        return pltpu.async_copy(table_hbm.at[pl.ds(row, CHUNK)], bufs[c % 2], gsem)

    gathers = {0: gather(0)}
    scatters = {}
    for c in range(NUM_CHUNKS):
        row = base + c * CHUNK
        gathers[c].wait()
        if c + 1 < NUM_CHUNKS:
            gathers[c + 1] = gather(c + 1)
        if c - 2 >= 0:
            for cp in scatters[c - 2]:
                cp.wait()
        scatters[c] = [
            pltpu.async_copy(bufs[c % 2], out_hbm.at[b, pl.ds(row, CHUNK)], ssem)
            for b in range(BATCH)
        ]
    for c in range(max(0, NUM_CHUNKS - 2), NUM_CHUNKS):
        for cp in scatters[c]:
            cp.wait()


def kernel(x, emb_weight):
    del x
    f = pl.kernel(
        _sc_broadcast,
        out_type=jax.ShapeDtypeStruct((BATCH, MAX_LEN, D_MODEL), jnp.float32),
        mesh=plsc.VectorSubcoreMesh(core_axis_name="c", subcore_axis_name="s"),
        scratch_types=[
            pltpu.VMEM((CHUNK, D_MODEL), jnp.float32),
            pltpu.VMEM((CHUNK, D_MODEL), jnp.float32),
            pltpu.SemaphoreType.DMA,
            pltpu.SemaphoreType.DMA,
        ],
    )
    return f(emb_weight)
